# baseline (speedup 1.0000x reference)
import jax
import jax.numpy as jnp
from jax import lax
from jax.experimental import pallas as pl
from jax.experimental.pallas import tpu as pltpu
from jax.experimental.pallas import tpu_sc as plsc

N = 100000
E = 1600000
EPS = 1e-3
EPS2 = EPS * EPS

NC, NS, L = 2, 16, 16
NW = NC * NS
PTS = N // NW
OFFS_BUF = 3152
OFFS_AL_MAX = ((N + 1 - OFFS_BUF) // 8) * 8
NZ_BUF = 3152
SEG = 4096
UNROLL = 4
SP_ROWS, SP_COLS = 784, 128

INT_MAX = 0x7FFFFFFF
MAGIC = 0x5F3759DF


def _softplus_body(x_ref, o_ref):
    z = 10.0 * x_ref[...]
    sp = jnp.log1p(jnp.exp(-jnp.abs(z))) + jnp.maximum(z, 0.0)
    o_ref[...] = sp * 0.1


def _sc_body(d_hbm, adj_hbm, offs_hbm, out_hbm,
             d_v, offs_v, nzs_v, nzr_v, adj_v, zs_v, acc_v,
             d_sem, seg_sem):
    cid = lax.axis_index("c")
    sid = lax.axis_index("s")
    wid = cid * NS + sid
    R0 = wid * PTS

    d_cp = pltpu.async_copy(d_hbm, d_v, d_sem)

    al = jnp.minimum(lax.bitwise_and(R0, -8), OFFS_AL_MAX)
    al = pl.multiple_of(al, 8)
    delta = R0 - al
    pltpu.sync_copy(offs_hbm.at[pl.ds(al, OFFS_BUF)], offs_v)

    iota = lax.iota(jnp.int32, L)
    delta_vec = jnp.full((L,), delta, jnp.int32)
    r0_vec = jnp.full((L,), R0, jnp.int32)

    ev0 = plsc.load_gather(offs_v, [delta_vec])
    ev1_raw = plsc.load_gather(
        offs_v, [jnp.minimum(delta_vec + PTS, OFFS_BUF - 1)])
    ev1 = jnp.where(r0_vec + PTS >= N, E, ev1_raw)
    e0al_vec = lax.bitwise_and(ev0, -8)
    e0al_s = jnp.max(e0al_vec)
    e1_s = jnp.max(ev1)
    n_seg = (e1_s - e0al_s + (SEG - 1)) // SEG

    def _start_seg(s, parity):
        start = pl.multiple_of(e0al_s + s * SEG, 8)
        boff = pl.multiple_of(parity * SEG, 8)
        return pltpu.async_copy(adj_hbm.at[pl.ds(start, SEG)],
                                adj_v.at[pl.ds(boff, SEG)],
                                seg_sem.at[parity])

    _start_seg(0, 0)

    def comp_body(c, p_vec):
        r = c * L + iota
        gi = delta_vec + r
        o_lo = plsc.load_gather(offs_v, [jnp.minimum(gi, OFFS_BUF - 1)])
        o_hi = plsc.load_gather(offs_v, [jnp.minimum(gi + 1, OFFS_BUF - 1)])
        o_hi = jnp.where(r0_vec + r >= N - 1, E, o_hi)
        m = jnp.logical_and(r < PTS, o_hi > o_lo)
        cs = plsc.cumsum(m.astype(jnp.int32))
        idx = p_vec + cs - 1
        plsc.store_scatter(nzs_v, [idx], o_lo, mask=m)
        plsc.store_scatter(nzr_v, [idx], r0_vec + r, mask=m)
        return p_vec + plsc.all_reduce_population_count(m)
    p_vec = lax.fori_loop(0, (PTS + 2 * L - 1) // L, comp_body,
                          jnp.zeros((L,), jnp.int32))
    plsc.store_scatter(nzs_v, [p_vec + iota],
                       jnp.full((L,), INT_MAX, jnp.int32))

    d_cp.wait()

    zv = jnp.zeros((L,), jnp.int32)

    def seg_body(s, carry):
        p = lax.bitwise_and(s, 1)
        _start_seg(s + 1, lax.bitwise_and(s + 1, 1))
        boff = pl.multiple_of(p * SEG, 8)
        pltpu.make_async_copy(adj_hbm.at[pl.ds(0, SEG)],
                              adj_v.at[pl.ds(boff, SEG)],
                              seg_sem.at[p]).wait()
        seg_raw = e0al_s + s * SEG
        seg_vec = e0al_vec + s * SEG
        rem = e1_s - seg_raw
        n_ch = jnp.minimum(SEG // L, jnp.maximum(0, (rem + (L - 1)) // L))
        n_grp = (n_ch + (UNROLL - 1)) // UNROLL
        base0 = p * SEG

        def grp_body(g, carry2):
            k_vec, acc = carry2
            for u in range(UNROLL):
                j16 = g * (UNROLL * L) + u * L
                eb = seg_vec + j16
                pos = eb + iota
                adjv = adj_v[pl.ds(pl.multiple_of(base0 + j16, 8), L)]
                w = k_vec + iota
                ws = plsc.load_gather(nzs_v, [w])
                wr = plsc.load_gather(nzr_v, [w])
                rel = ws - eb
                m = plsc.bitcast(rel, jnp.uint32) < L
                zr = zs_v.at[pl.ds(u * L, L)]
                zr[...] = zv
                plsc.store_scatter(zr, [rel], wr, mask=m)
                cr = plsc.load_gather(nzr_v, [jnp.maximum(k_vec - 1, 0)])
                cz = plsc.cummax(plsc.bitcast(zr[...], jnp.uint32))
                src = plsc.bitcast(
                    jnp.maximum(cz, plsc.bitcast(cr, jnp.uint32)), jnp.int32)
                dn = plsc.load_gather(d_v, [adjv])
                ds = plsc.load_gather(d_v, [src])
                diff = ds - dn
                q = diff * diff + EPS2
                yi = MAGIC - lax.shift_right_logical(
                    plsc.bitcast(q, jnp.int32), 1)
                y = plsc.bitcast(yi, jnp.float32)
                y = y * (1.5 - 0.5 * q * y * y)
                s_val = q * y
                valid = jnp.logical_and(pos >= ev0, pos < ev1)
                acc = acc + jnp.where(valid, s_val - EPS, 0.0)
                k_vec = k_vec + plsc.all_reduce_population_count(m)
            return (k_vec, acc)

        return lax.fori_loop(0, n_grp, grp_body, carry)

    _, acc = lax.fori_loop(
        0, n_seg, seg_body,
        (jnp.zeros((L,), jnp.int32), jnp.zeros((L,), jnp.float32)))
    fin = pl.multiple_of(lax.bitwise_and(n_seg, 1) * SEG, 8)
    pltpu.make_async_copy(adj_hbm.at[pl.ds(0, SEG)],
                          adj_v.at[pl.ds(fin, SEG)],
                          seg_sem.at[lax.bitwise_and(n_seg, 1)]).wait()
    acc_v[...] = acc
    pltpu.sync_copy(acc_v, out_hbm.at[wid])


_sc_kernel = pl.kernel(
    _sc_body,
    out_type=jax.ShapeDtypeStruct((NW, L), jnp.float32),
    mesh=plsc.VectorSubcoreMesh(core_axis_name="c", subcore_axis_name="s",
                                num_cores=NC, num_subcores=NS),
    compiler_params=pltpu.CompilerParams(needs_layout_passes=False),
    scratch_types=[
        pltpu.VMEM((N,), jnp.float32),
        pltpu.VMEM((OFFS_BUF,), jnp.int32),
        pltpu.VMEM((NZ_BUF,), jnp.int32),
        pltpu.VMEM((NZ_BUF,), jnp.int32),
        pltpu.VMEM((2 * SEG,), jnp.int32),
        pltpu.VMEM((UNROLL * L,), jnp.int32),
        pltpu.VMEM((L,), jnp.float32),
        pltpu.SemaphoreType.DMA,
        pltpu.SemaphoreType.DMA((2,)),
    ],
)


def kernel(density, point_adjacency, point_adjacency_offsets):
    dpad = jnp.pad(density[:, 0], (0, SP_ROWS * SP_COLS - N))
    d2 = pl.pallas_call(
        _softplus_body,
        out_shape=jax.ShapeDtypeStruct((SP_ROWS, SP_COLS), jnp.float32),
    )(dpad.reshape(SP_ROWS, SP_COLS))
    d_flat = d2.reshape(-1)[:N]
    adj = jnp.pad(point_adjacency.astype(jnp.int32), (0, 2 * SEG))
    partial = _sc_kernel(d_flat, adj,
                         point_adjacency_offsets.astype(jnp.int32))
    return jnp.sum(partial) / E


# speedup vs baseline: 1.0709x; 1.0709x over previous
import jax
import jax.numpy as jnp
from jax import lax
from jax.experimental import pallas as pl
from jax.experimental.pallas import tpu as pltpu
from jax.experimental.pallas import tpu_sc as plsc

N = 100000
E = 1600000
EPS = 1e-3
EPS2 = EPS * EPS

NC, NS, L = 2, 16, 16
NW = NC * NS
PTS = N // NW
OFFS_BUF = 3152
OFFS_AL_MAX = ((N + 1 - OFFS_BUF) // 8) * 8
NZ_BUF = 3152
SEG = 4096
SEG_MAX = E - SEG
UNROLL = 4
SP_ROWS, SP_COLS = 784, 128

INT_MAX = 0x7FFFFFFF
MAGIC = 0x5F3759DF


def _softplus_body(x_ref, o_ref):
    z = 10.0 * x_ref[...]
    sp = jnp.log1p(jnp.exp(-jnp.abs(z))) + jnp.maximum(z, 0.0)
    o_ref[...] = sp * 0.1


def _sc_body(d_hbm, adj_hbm, offs_hbm, out_hbm,
             d_v, offs_v, nzs_v, nzr_v, adj_v, zs_v, acc_v,
             d_sem, seg_sem):
    cid = lax.axis_index("c")
    sid = lax.axis_index("s")
    wid = cid * NS + sid
    R0 = wid * PTS

    d_cp = pltpu.async_copy(d_hbm, d_v, d_sem)

    al = jnp.minimum(lax.bitwise_and(R0, -8), OFFS_AL_MAX)
    al = pl.multiple_of(al, 8)
    delta = R0 - al
    pltpu.sync_copy(offs_hbm.at[pl.ds(al, OFFS_BUF)], offs_v)

    iota = lax.iota(jnp.int32, L)
    delta_vec = jnp.full((L,), delta, jnp.int32)
    r0_vec = jnp.full((L,), R0, jnp.int32)

    ev0 = plsc.load_gather(offs_v, [delta_vec])
    ev1_raw = plsc.load_gather(
        offs_v, [jnp.minimum(delta_vec + PTS, OFFS_BUF - 1)])
    ev1 = jnp.where(r0_vec + PTS >= N, E, ev1_raw)
    e0al_vec = lax.bitwise_and(ev0, -8)
    e0al_s = jnp.max(e0al_vec)
    e1_s = jnp.max(ev1)
    n_seg = (e1_s - e0al_s + (SEG - 1)) // SEG

    def _start_seg(s, parity):
        start = jnp.minimum(e0al_s + s * SEG, SEG_MAX)
        start = pl.multiple_of(start, 8)
        boff = pl.multiple_of(parity * SEG, 8)
        return pltpu.async_copy(adj_hbm.at[pl.ds(start, SEG)],
                                adj_v.at[pl.ds(boff, SEG)],
                                seg_sem.at[parity])

    _start_seg(0, 0)

    def comp_body(c, p_vec):
        r = c * L + iota
        gi = delta_vec + r
        o_lo = plsc.load_gather(offs_v, [jnp.minimum(gi, OFFS_BUF - 1)])
        o_hi = plsc.load_gather(offs_v, [jnp.minimum(gi + 1, OFFS_BUF - 1)])
        o_hi = jnp.where(r0_vec + r >= N - 1, E, o_hi)
        m = jnp.logical_and(r < PTS, o_hi > o_lo)
        cs = plsc.cumsum(m.astype(jnp.int32))
        idx = p_vec + cs - 1
        plsc.store_scatter(nzs_v, [idx], o_lo, mask=m)
        plsc.store_scatter(nzr_v, [idx], r0_vec + r, mask=m)
        return p_vec + plsc.all_reduce_population_count(m)
    p_vec = lax.fori_loop(0, (PTS + 2 * L - 1) // L, comp_body,
                          jnp.zeros((L,), jnp.int32))
    plsc.store_scatter(nzs_v, [p_vec + iota],
                       jnp.full((L,), INT_MAX, jnp.int32))

    d_cp.wait()

    zv = jnp.zeros((L,), jnp.int32)

    def seg_body(s, carry):
        p = lax.bitwise_and(s, 1)
        _start_seg(s + 1, lax.bitwise_and(s + 1, 1))
        boff = pl.multiple_of(p * SEG, 8)
        pltpu.make_async_copy(adj_hbm.at[pl.ds(0, SEG)],
                              adj_v.at[pl.ds(boff, SEG)],
                              seg_sem.at[p]).wait()
        seg_raw = e0al_s + s * SEG
        shift = seg_raw - jnp.minimum(seg_raw, SEG_MAX)
        seg_vec = e0al_vec + s * SEG
        rem = e1_s - seg_raw
        n_ch = jnp.minimum(SEG // L, jnp.maximum(0, (rem + (L - 1)) // L))
        n_grp = (n_ch + (UNROLL - 1)) // UNROLL
        clamp_hi = p * SEG + (SEG - 1)
        base0 = p * SEG + shift

        def grp_body(g, carry2):
            k_vec, acc = carry2
            for u in range(UNROLL):
                j16 = g * (UNROLL * L) + u * L
                eb = seg_vec + j16
                pos = eb + iota
                aidx = jnp.minimum(base0 + j16 + iota, clamp_hi)
                adjv = plsc.load_gather(adj_v, [aidx])
                w = k_vec + iota
                ws = plsc.load_gather(nzs_v, [w])
                wr = plsc.load_gather(nzr_v, [w])
                rel = ws - eb
                m = plsc.bitcast(rel, jnp.uint32) < L
                zr = zs_v.at[pl.ds(u * L, L)]
                zr[...] = zv
                plsc.store_scatter(zr, [rel], wr, mask=m)
                cr = plsc.load_gather(nzr_v, [jnp.maximum(k_vec - 1, 0)])
                cz = plsc.cummax(plsc.bitcast(zr[...], jnp.uint32))
                src = plsc.bitcast(
                    jnp.maximum(cz, plsc.bitcast(cr, jnp.uint32)), jnp.int32)
                dn = plsc.load_gather(d_v, [adjv])
                ds = plsc.load_gather(d_v, [src])
                diff = ds - dn
                q = diff * diff + EPS2
                yi = MAGIC - lax.shift_right_logical(
                    plsc.bitcast(q, jnp.int32), 1)
                y = plsc.bitcast(yi, jnp.float32)
                y = y * (1.5 - 0.5 * q * y * y)
                s_val = q * y
                valid = jnp.logical_and(pos >= ev0, pos < ev1)
                acc = acc + jnp.where(valid, s_val - EPS, 0.0)
                k_vec = k_vec + plsc.all_reduce_population_count(m)
            return (k_vec, acc)

        return lax.fori_loop(0, n_grp, grp_body, carry)

    _, acc = lax.fori_loop(
        0, n_seg, seg_body,
        (jnp.zeros((L,), jnp.int32), jnp.zeros((L,), jnp.float32)))
    fin = pl.multiple_of(lax.bitwise_and(n_seg, 1) * SEG, 8)
    pltpu.make_async_copy(adj_hbm.at[pl.ds(0, SEG)],
                          adj_v.at[pl.ds(fin, SEG)],
                          seg_sem.at[lax.bitwise_and(n_seg, 1)]).wait()
    acc_v[...] = acc
    pltpu.sync_copy(acc_v, out_hbm.at[wid])


_sc_kernel = pl.kernel(
    _sc_body,
    out_type=jax.ShapeDtypeStruct((NW, L), jnp.float32),
    mesh=plsc.VectorSubcoreMesh(core_axis_name="c", subcore_axis_name="s",
                                num_cores=NC, num_subcores=NS),
    compiler_params=pltpu.CompilerParams(needs_layout_passes=False),
    scratch_types=[
        pltpu.VMEM((N,), jnp.float32),
        pltpu.VMEM((OFFS_BUF,), jnp.int32),
        pltpu.VMEM((NZ_BUF,), jnp.int32),
        pltpu.VMEM((NZ_BUF,), jnp.int32),
        pltpu.VMEM((2 * SEG,), jnp.int32),
        pltpu.VMEM((UNROLL * L,), jnp.int32),
        pltpu.VMEM((L,), jnp.float32),
        pltpu.SemaphoreType.DMA,
        pltpu.SemaphoreType.DMA((2,)),
    ],
)


def kernel(density, point_adjacency, point_adjacency_offsets):
    dpad = jnp.pad(density[:, 0], (0, SP_ROWS * SP_COLS - N))
    d2 = pl.pallas_call(
        _softplus_body,
        out_shape=jax.ShapeDtypeStruct((SP_ROWS, SP_COLS), jnp.float32),
    )(dpad.reshape(SP_ROWS, SP_COLS))
    d_flat = d2.reshape(-1)[:N]
    partial = _sc_kernel(d_flat, point_adjacency.astype(jnp.int32),
                         point_adjacency_offsets.astype(jnp.int32))
    return jnp.sum(partial) / E
